# baseline (speedup 1.0000x reference)
import jax
import jax.numpy as jnp
from jax.experimental import pallas as pl
from jax.experimental.pallas import tpu as pltpu

_B = 2
_Q = 1024
_N = 100000
_D = 64
_K = 16
_C = 2048
_NCHUNK = 49
_NPAD = _C * _NCHUNK
_NBIN = _NCHUNK * 8


def _body(r_ref, qt_ref, q2_ref, r2_ref, oidx_ref, oval_ref, accv, acci):
    c = pl.program_id(1)

    rb = r_ref[0]
    qt = qt_ref[0]
    q2 = q2_ref[0]
    r2 = r2_ref[0]

    xy = jax.lax.dot_general(rb, qt, (((1,), (0,)), ((), ())),
                             preferred_element_type=jnp.float32)
    d = (r2 + q2) - 2.0 * xy

    v3 = d.reshape(_C // 8, 8, _Q)
    cls_v = jnp.min(v3, axis=0)
    ii = jax.lax.broadcasted_iota(jnp.int32, (_C // 8, 8, _Q), 0)
    win_a = jnp.max(jnp.where(v3 == cls_v[None], ii, -1), axis=0)
    cls_c = jax.lax.broadcasted_iota(jnp.int32, (8, _Q), 0)
    win_i = c * _C + win_a * 8 + cls_c

    accv[pl.ds(c * 8, 8), :] = cls_v
    acci[pl.ds(c * 8, 8), :] = win_i

    @pl.when(c == _NCHUNK - 1)
    def _extract():
        vals = accv[...]
        wins = acci[...]
        pos = jax.lax.broadcasted_iota(jnp.int32, (_NBIN, _Q), 0)
        idx_rows = []
        val_rows = []
        for _ in range(_K):
            colmin = jnp.min(vals, axis=0, keepdims=True)
            eq = vals == colmin
            selpos = jnp.min(jnp.where(eq, pos, _NBIN), axis=0,
                             keepdims=True)
            onehot = pos == selpos
            isel = jnp.max(jnp.where(onehot, wins, -1), axis=0,
                           keepdims=True)
            idx_rows.append(isel)
            val_rows.append(colmin)
            vals = jnp.where(onehot, jnp.inf, vals)
        oidx_ref[0] = jnp.concatenate(idx_rows, axis=0)
        oval_ref[0] = jnp.concatenate(val_rows, axis=0)


def _sumsq64(x):
    p = x * x
    v = p.reshape(p.shape[:-1] + (8, 8))
    acc = v[..., 0, :]
    for k in range(1, 8):
        acc = acc + v[..., k, :]
    a = acc[..., :4] + acc[..., 4:]
    b = a[..., :2] + a[..., 2:]
    return b[..., 0] + b[..., 1]


def kernel(q, r):
    q = jnp.asarray(q, jnp.float32)
    r = jnp.asarray(r, jnp.float32)
    rp = jnp.pad(r, ((0, 0), (0, _NPAD - _N), (0, 0)))
    qt = jnp.transpose(q, (0, 2, 1)).astype(jnp.bfloat16)
    q2 = _sumsq64(q)[:, None, :]
    r2 = _sumsq64(rp)
    r2 = jnp.where(jnp.arange(_NPAD)[None, :] < _N, r2, jnp.inf)
    r2 = r2.reshape(_B * _NCHUNK, _C, 1)

    rpb = rp.astype(jnp.bfloat16)

    out_idx, out_val = pl.pallas_call(
        _body,
        grid=(_B, _NCHUNK),
        in_specs=[
            pl.BlockSpec((1, _C, _D), lambda a, c: (a, c, 0)),
            pl.BlockSpec((1, _D, _Q), lambda a, c: (a, 0, 0)),
            pl.BlockSpec((1, 1, _Q), lambda a, c: (a, 0, 0)),
            pl.BlockSpec((1, _C, 1), lambda a, c: (a * _NCHUNK + c, 0, 0)),
        ],
        out_specs=[
            pl.BlockSpec((1, _K, _Q), lambda a, c: (a, 0, 0)),
            pl.BlockSpec((1, _K, _Q), lambda a, c: (a, 0, 0)),
        ],
        out_shape=[
            jax.ShapeDtypeStruct((_B, _K, _Q), jnp.int32),
            jax.ShapeDtypeStruct((_B, _K, _Q), jnp.float32),
        ],
        scratch_shapes=[
            pltpu.VMEM((_NBIN, _Q), jnp.float32),
            pltpu.VMEM((_NBIN, _Q), jnp.int32),
        ],
        compiler_params=pltpu.CompilerParams(
            dimension_semantics=("arbitrary", "arbitrary"),
        ),
    )(rpb, qt, q2, r2)
    return (jnp.transpose(out_idx, (0, 2, 1)),
            jnp.transpose(out_val, (0, 2, 1)))


# speedup vs baseline: 1.0027x; 1.0027x over previous
import jax
import jax.numpy as jnp
from jax.experimental import pallas as pl
from jax.experimental.pallas import tpu as pltpu

_B = 2
_Q = 1024
_N = 100000
_D = 64
_K = 16
_C = 2048
_NCHUNK = 49
_NPAD = _C * _NCHUNK
_NBIN = _NCHUNK * 8


def _body(r_ref, qt_ref, q2_ref, r2_ref, oidx_ref, oval_ref, accv, acci):
    c = pl.program_id(1)

    rb = r_ref[0]
    qt = qt_ref[0]
    q2 = q2_ref[0]
    r2 = r2_ref[0]

    xy = jax.lax.dot_general(rb, qt, (((1,), (0,)), ((), ())),
                             preferred_element_type=jnp.float32)
    d = (r2 + q2) - 2.0 * xy

    v = d
    iv = jax.lax.broadcasted_iota(jnp.int32, (_C, _Q), 0)
    h = _C // 2
    while h >= 8:
        av, bv = v[:h], v[h:]
        ai, bi = iv[:h], iv[h:]
        sel = (av < bv) | ((av == bv) & (ai > bi))
        v = jnp.where(sel, av, bv)
        iv = jnp.where(sel, ai, bi)
        h //= 2
    accv[pl.ds(c * 8, 8), :] = v
    acci[pl.ds(c * 8, 8), :] = c * _C + iv

    @pl.when(c == _NCHUNK - 1)
    def _extract():
        vals = accv[...]
        wins = acci[...]
        pos = jax.lax.broadcasted_iota(jnp.int32, (_NBIN, _Q), 0)
        idx_rows = []
        val_rows = []
        for _ in range(_K):
            colmin = jnp.min(vals, axis=0, keepdims=True)
            eq = vals == colmin
            selpos = jnp.min(jnp.where(eq, pos, _NBIN), axis=0,
                             keepdims=True)
            onehot = pos == selpos
            isel = jnp.max(jnp.where(onehot, wins, -1), axis=0,
                           keepdims=True)
            idx_rows.append(isel)
            val_rows.append(colmin)
            vals = jnp.where(onehot, jnp.inf, vals)
        oidx_ref[0] = jnp.concatenate(idx_rows, axis=0)
        oval_ref[0] = jnp.concatenate(val_rows, axis=0)


def _sumsq64(x):
    p = x * x
    v = p.reshape(p.shape[:-1] + (8, 8))
    acc = v[..., 0, :]
    for k in range(1, 8):
        acc = acc + v[..., k, :]
    a = acc[..., :4] + acc[..., 4:]
    b = a[..., :2] + a[..., 2:]
    return b[..., 0] + b[..., 1]


def kernel(q, r):
    q = jnp.asarray(q, jnp.float32)
    r = jnp.asarray(r, jnp.float32)
    rp = jnp.pad(r, ((0, 0), (0, _NPAD - _N), (0, 0)))
    qt = jnp.transpose(q, (0, 2, 1)).astype(jnp.bfloat16)
    q2 = _sumsq64(q)[:, None, :]
    r2 = _sumsq64(rp)
    r2 = jnp.where(jnp.arange(_NPAD)[None, :] < _N, r2, jnp.inf)
    r2 = r2.reshape(_B * _NCHUNK, _C, 1)

    rpb = rp.astype(jnp.bfloat16)

    out_idx, out_val = pl.pallas_call(
        _body,
        grid=(_B, _NCHUNK),
        in_specs=[
            pl.BlockSpec((1, _C, _D), lambda a, c: (a, c, 0)),
            pl.BlockSpec((1, _D, _Q), lambda a, c: (a, 0, 0)),
            pl.BlockSpec((1, 1, _Q), lambda a, c: (a, 0, 0)),
            pl.BlockSpec((1, _C, 1), lambda a, c: (a * _NCHUNK + c, 0, 0)),
        ],
        out_specs=[
            pl.BlockSpec((1, _K, _Q), lambda a, c: (a, 0, 0)),
            pl.BlockSpec((1, _K, _Q), lambda a, c: (a, 0, 0)),
        ],
        out_shape=[
            jax.ShapeDtypeStruct((_B, _K, _Q), jnp.int32),
            jax.ShapeDtypeStruct((_B, _K, _Q), jnp.float32),
        ],
        scratch_shapes=[
            pltpu.VMEM((_NBIN, _Q), jnp.float32),
            pltpu.VMEM((_NBIN, _Q), jnp.int32),
        ],
        compiler_params=pltpu.CompilerParams(
            dimension_semantics=("arbitrary", "arbitrary"),
        ),
    )(rpb, qt, q2, r2)
    return (jnp.transpose(out_idx, (0, 2, 1)),
            jnp.transpose(out_val, (0, 2, 1)))


# speedup vs baseline: 1.2397x; 1.2364x over previous
import jax
import jax.numpy as jnp
from jax.experimental import pallas as pl
from jax.experimental.pallas import tpu as pltpu

_B = 2
_Q = 1024
_N = 100000
_D = 64
_K = 16
_C = 2048
_NCHUNK = 49
_NPAD = _C * _NCHUNK
_NBIN = _NCHUNK * 8


def _body(r_ref, qt_ref, q2_ref, r2_ref, oidx_ref, oval_ref, accv, acci):
    c = pl.program_id(1)

    rb = r_ref[0]
    qt = qt_ref[0]
    q2 = q2_ref[0]
    r2 = r2_ref[0]

    xy = jax.lax.dot_general(rb, qt, (((1,), (0,)), ((), ())),
                             preferred_element_type=jnp.float32)
    d = (r2 + q2) - 2.0 * xy

    v = d
    iv = jax.lax.broadcasted_iota(jnp.int32, (_C, _Q), 0)
    h = _C // 2
    while h >= 8:
        av, bv = v[:h], v[h:]
        ai, bi = iv[:h], iv[h:]
        sel = (av < bv) | ((av == bv) & (ai > bi))
        v = jnp.where(sel, av, bv)
        iv = jnp.where(sel, ai, bi)
        h //= 2
    accv[pl.ds(c * 8, 8), :] = v
    acci[pl.ds(c * 8, 8), :] = c * _C + iv

    @pl.when(c == _NCHUNK - 1)
    def _extract():
        vals = accv[...]
        wins = acci[...]
        pos = jax.lax.broadcasted_iota(jnp.int32, (_NBIN, _Q), 0)
        idx_rows = []
        val_rows = []
        for _ in range(_K):
            colmin = jnp.min(vals, axis=0, keepdims=True)
            eq = vals == colmin
            selpos = jnp.min(jnp.where(eq, pos, _NBIN), axis=0,
                             keepdims=True)
            onehot = pos == selpos
            isel = jnp.max(jnp.where(onehot, wins, -1), axis=0,
                           keepdims=True)
            idx_rows.append(isel)
            val_rows.append(colmin)
            vals = jnp.where(onehot, jnp.inf, vals)
        oidx_ref[0] = jnp.concatenate(idx_rows, axis=0)
        oval_ref[0] = jnp.concatenate(val_rows, axis=0)


def _sumsq64(x):
    p = x * x
    v = p.reshape(p.shape[:-1] + (8, 8))
    acc = v[..., 0, :]
    for k in range(1, 8):
        acc = acc + v[..., k, :]
    a = acc[..., :4] + acc[..., 4:]
    b = a[..., :2] + a[..., 2:]
    return b[..., 0] + b[..., 1]


def kernel(q, r):
    q = jnp.asarray(q, jnp.float32)
    r = jnp.asarray(r, jnp.float32)
    qt = jnp.transpose(q, (0, 2, 1)).astype(jnp.bfloat16)
    q2 = _sumsq64(q)[:, None, :]
    r2 = jnp.pad(_sumsq64(r), ((0, 0), (0, _NPAD - _N)),
                 constant_values=jnp.inf)
    r2 = r2.reshape(_B * _NCHUNK, _C, 1)

    rpb = jnp.pad(r.astype(jnp.bfloat16), ((0, 0), (0, _NPAD - _N), (0, 0)))

    out_idx, out_val = pl.pallas_call(
        _body,
        grid=(_B, _NCHUNK),
        in_specs=[
            pl.BlockSpec((1, _C, _D), lambda a, c: (a, c, 0)),
            pl.BlockSpec((1, _D, _Q), lambda a, c: (a, 0, 0)),
            pl.BlockSpec((1, 1, _Q), lambda a, c: (a, 0, 0)),
            pl.BlockSpec((1, _C, 1), lambda a, c: (a * _NCHUNK + c, 0, 0)),
        ],
        out_specs=[
            pl.BlockSpec((1, _K, _Q), lambda a, c: (a, 0, 0)),
            pl.BlockSpec((1, _K, _Q), lambda a, c: (a, 0, 0)),
        ],
        out_shape=[
            jax.ShapeDtypeStruct((_B, _K, _Q), jnp.int32),
            jax.ShapeDtypeStruct((_B, _K, _Q), jnp.float32),
        ],
        scratch_shapes=[
            pltpu.VMEM((_NBIN, _Q), jnp.float32),
            pltpu.VMEM((_NBIN, _Q), jnp.int32),
        ],
        compiler_params=pltpu.CompilerParams(
            dimension_semantics=("arbitrary", "arbitrary"),
        ),
    )(rpb, qt, q2, r2)
    return (jnp.transpose(out_idx, (0, 2, 1)),
            jnp.transpose(out_val, (0, 2, 1)))


# speedup vs baseline: 1.2488x; 1.0073x over previous
import jax
import jax.numpy as jnp
from jax.experimental import pallas as pl
from jax.experimental.pallas import tpu as pltpu

_B = 2
_Q = 1024
_N = 100000
_D = 64
_K = 16
_C = 4096
_NCHUNK = 25
_NPAD = _C * _NCHUNK
_NBIN = _NCHUNK * 16


def _body(r_ref, qt_ref, q2_ref, r2_ref, oidx_ref, oval_ref, accv, acci):
    c = pl.program_id(1)

    rb = r_ref[0]
    qt = qt_ref[0]
    q2 = q2_ref[0]
    r2 = r2_ref[0]

    xy = jax.lax.dot_general(rb, qt, (((1,), (0,)), ((), ())),
                             preferred_element_type=jnp.float32)
    d = (r2 + q2) - 2.0 * xy

    for g in range(2):
        v = d[g * 2048:(g + 1) * 2048]
        iv = jax.lax.broadcasted_iota(jnp.int32, (2048, _Q), 0)
        h = 1024
        while h >= 8:
            av, bv = v[:h], v[h:]
            ai, bi = iv[:h], iv[h:]
            sel = (av < bv) | ((av == bv) & (ai > bi))
            v = jnp.where(sel, av, bv)
            iv = jnp.where(sel, ai, bi)
            h //= 2
        accv[pl.ds(c * 16 + g * 8, 8), :] = v
        acci[pl.ds(c * 16 + g * 8, 8), :] = c * _C + g * 2048 + iv

    @pl.when(c == _NCHUNK - 1)
    def _extract():
        vals = accv[...]
        wins = acci[...]
        pos = jax.lax.broadcasted_iota(jnp.int32, (_NBIN, _Q), 0)
        idx_rows = []
        val_rows = []
        for _ in range(_K):
            colmin = jnp.min(vals, axis=0, keepdims=True)
            eq = vals == colmin
            selpos = jnp.min(jnp.where(eq, pos, _NBIN), axis=0,
                             keepdims=True)
            onehot = pos == selpos
            isel = jnp.max(jnp.where(onehot, wins, -1), axis=0,
                           keepdims=True)
            idx_rows.append(isel)
            val_rows.append(colmin)
            vals = jnp.where(onehot, jnp.inf, vals)
        oidx_ref[0] = jnp.concatenate(idx_rows, axis=0)
        oval_ref[0] = jnp.concatenate(val_rows, axis=0)


def _sumsq64(x):
    p = x * x
    v = p.reshape(p.shape[:-1] + (8, 8))
    acc = v[..., 0, :]
    for k in range(1, 8):
        acc = acc + v[..., k, :]
    a = acc[..., :4] + acc[..., 4:]
    b = a[..., :2] + a[..., 2:]
    return b[..., 0] + b[..., 1]


def kernel(q, r):
    q = jnp.asarray(q, jnp.float32)
    r = jnp.asarray(r, jnp.float32)
    qt = jnp.transpose(q, (0, 2, 1)).astype(jnp.bfloat16)
    q2 = _sumsq64(q)[:, None, :]
    r2 = jnp.pad(_sumsq64(r), ((0, 0), (0, _NPAD - _N)),
                 constant_values=jnp.inf)
    r2 = r2.reshape(_B * _NCHUNK, _C, 1)

    rpb = jnp.pad(r.astype(jnp.bfloat16), ((0, 0), (0, _NPAD - _N), (0, 0)))

    out_idx, out_val = pl.pallas_call(
        _body,
        grid=(_B, _NCHUNK),
        in_specs=[
            pl.BlockSpec((1, _C, _D), lambda a, c: (a, c, 0)),
            pl.BlockSpec((1, _D, _Q), lambda a, c: (a, 0, 0)),
            pl.BlockSpec((1, 1, _Q), lambda a, c: (a, 0, 0)),
            pl.BlockSpec((1, _C, 1), lambda a, c: (a * _NCHUNK + c, 0, 0)),
        ],
        out_specs=[
            pl.BlockSpec((1, _K, _Q), lambda a, c: (a, 0, 0)),
            pl.BlockSpec((1, _K, _Q), lambda a, c: (a, 0, 0)),
        ],
        out_shape=[
            jax.ShapeDtypeStruct((_B, _K, _Q), jnp.int32),
            jax.ShapeDtypeStruct((_B, _K, _Q), jnp.float32),
        ],
        scratch_shapes=[
            pltpu.VMEM((_NBIN, _Q), jnp.float32),
            pltpu.VMEM((_NBIN, _Q), jnp.int32),
        ],
        compiler_params=pltpu.CompilerParams(
            dimension_semantics=("arbitrary", "arbitrary"),
        ),
    )(rpb, qt, q2, r2)
    return (jnp.transpose(out_idx, (0, 2, 1)),
            jnp.transpose(out_val, (0, 2, 1)))
